# baseline (speedup 1.0000x reference)
import functools

import jax
import jax.numpy as jnp
from jax import lax
from jax.experimental import pallas as pl
from jax.experimental.pallas import tpu as pltpu
from jax.experimental.pallas import tpu_sc as plsc

D_MODEL = 128
SUB = 128
NBUF = 5


def _gather_sc(n_total: int):
    info = plsc.get_sparse_core_info()
    nw = info.num_cores * info.num_subcores
    n_per_w = n_total // nw
    n_sub = n_per_w // SUB
    assert n_sub % NBUF == 0
    n_outer = n_sub // NBUF

    mesh = plsc.VectorSubcoreMesh(core_axis_name="c", subcore_axis_name="s")

    @functools.partial(
        pl.kernel,
        mesh=mesh,
        out_type=jax.ShapeDtypeStruct((n_total, D_MODEL), jnp.float32),
        scratch_types=[
            pltpu.VMEM((n_sub, SUB), jnp.int32),
            pltpu.VMEM((NBUF, SUB, D_MODEL), jnp.float32),
            pltpu.VMEM_SHARED((366, D_MODEL), jnp.float32),
        ] + [pltpu.SemaphoreType.DMA] * (2 * NBUF),
    )
    def gather_kernel(idx_hbm, table_hbm, out_hbm, idx_v, rows, table_sh,
                      *sems):
        gsems = sems[:NBUF]
        ssems = sems[NBUF:]
        wid = lax.axis_index("s") * info.num_cores + lax.axis_index("c")
        base = wid * n_per_w

        @pl.when(lax.axis_index("s") == 0)
        def _():
            pltpu.sync_copy(table_hbm, table_sh)
        plsc.subcore_barrier()

        pltpu.sync_copy(idx_hbm.at[wid], idx_v)

        def body(io, carry):
            for b in range(NBUF):
                i = io * NBUF + b

                @pl.when(io > 0)
                def _():
                    pltpu.make_async_copy(
                        rows.at[b], out_hbm.at[pl.ds(base, SUB)],
                        ssems[b]).wait()

                pltpu.async_copy(
                    table_sh.at[idx_v.at[i]], rows.at[b], gsems[b])
            for b in range(NBUF):
                i = io * NBUF + b
                pltpu.make_async_copy(
                    table_sh.at[idx_v.at[i]], rows.at[b], gsems[b]).wait()
                pltpu.async_copy(
                    rows.at[b], out_hbm.at[pl.ds(base + i * SUB, SUB)],
                    ssems[b])
            return carry

        lax.fori_loop(0, n_outer, body, 0)

        for b in range(NBUF):
            pltpu.make_async_copy(
                rows.at[b], out_hbm.at[pl.ds(base, SUB)], ssems[b]).wait()

    return gather_kernel


def kernel(positions, pe):
    b, t = positions.shape
    n_total = b * t
    info = plsc.get_sparse_core_info()
    nw = info.num_cores * info.num_subcores
    idx = positions.reshape(nw, (n_total // nw) // SUB, SUB).astype(jnp.int32)
    out = _gather_sc(n_total)(idx, pe)
    return out.reshape(b, t, D_MODEL)


# speedup vs baseline: 1.0116x; 1.0116x over previous
import functools

import jax
import jax.numpy as jnp
from jax import lax
from jax.experimental import pallas as pl
from jax.experimental.pallas import tpu as pltpu
from jax.experimental.pallas import tpu_sc as plsc

D_MODEL = 128
SUB = 128
NBUF = 4


def _gather_sc(n_total: int):
    info = plsc.get_sparse_core_info()
    nw = info.num_cores * info.num_subcores
    n_per_w = n_total // nw
    n_sub = n_per_w // SUB
    assert n_sub % NBUF == 0
    n_outer = n_sub // NBUF

    mesh = plsc.VectorSubcoreMesh(core_axis_name="c", subcore_axis_name="s")

    @functools.partial(
        pl.kernel,
        mesh=mesh,
        out_type=jax.ShapeDtypeStruct((n_total, D_MODEL), jnp.float32),
        scratch_types=[
            pltpu.VMEM((n_sub, SUB), jnp.int32),
            pltpu.VMEM((NBUF, SUB, D_MODEL), jnp.float32),
            pltpu.VMEM_SHARED((366, D_MODEL), jnp.float32),
        ] + [pltpu.SemaphoreType.DMA] * (2 * NBUF + 1),
    )
    def gather_kernel(idx_hbm, table_hbm, out_hbm, idx_v, rows, table_sh,
                      *sems):
        gsems = sems[:NBUF]
        ssems = sems[NBUF : 2 * NBUF]
        isem = sems[2 * NBUF]
        wid = lax.axis_index("s") * info.num_cores + lax.axis_index("c")
        base = wid * n_per_w

        idx_copy = pltpu.async_copy(idx_hbm.at[wid], idx_v, isem)

        @pl.when(lax.axis_index("s") == 0)
        def _():
            pltpu.sync_copy(table_hbm, table_sh)
        plsc.subcore_barrier()
        idx_copy.wait()

        def body(io, carry):
            for b in range(NBUF):
                i = io * NBUF + b

                @pl.when(io > 0)
                def _():
                    pltpu.make_async_copy(
                        rows.at[b], out_hbm.at[pl.ds(base, SUB)],
                        ssems[b]).wait()

                pltpu.async_copy(
                    table_sh.at[idx_v.at[i]], rows.at[b], gsems[b])
            for b in range(NBUF):
                i = io * NBUF + b
                pltpu.make_async_copy(
                    table_sh.at[idx_v.at[i]], rows.at[b], gsems[b]).wait()
                pltpu.async_copy(
                    rows.at[b], out_hbm.at[pl.ds(base + i * SUB, SUB)],
                    ssems[b])
            return carry

        lax.fori_loop(0, n_outer, body, 0)

        for b in range(NBUF):
            pltpu.make_async_copy(
                rows.at[b], out_hbm.at[pl.ds(base, SUB)], ssems[b]).wait()

    return gather_kernel


def kernel(positions, pe):
    b, t = positions.shape
    n_total = b * t
    info = plsc.get_sparse_core_info()
    nw = info.num_cores * info.num_subcores
    idx = positions.reshape(nw, (n_total // nw) // SUB, SUB).astype(jnp.int32)
    out = _gather_sc(n_total)(idx, pe)
    return out.reshape(b, t, D_MODEL)
